# initial kernel scaffold (unmeasured)
import jax
import jax.numpy as jnp
from jax import lax
from jax.experimental import pallas as pl
from jax.experimental.pallas import tpu as pltpu

N_DEV = 4
B, H, D, BS = 16, 16, 64, 16
NB = 128
PAGES_LOCAL = 128
KEYS_LOCAL = PAGES_LOCAL * BS
NEG_INF = -1e30


def kernel(Q, K, V, bt, lens):
    lens2 = lens.reshape(B, 1).astype(jnp.int32)

    def body(q_ref, k_ref, v_ref, bt_ref, lens_ref, out_ref,
             comm_ref, send_sems, recv_sems):
        my_i = lax.axis_index("i")

        bt3 = bt_ref[:, :][:, :, None]
        lens3 = lens_ref[:, :][:, :, None]
        kpos = lax.broadcasted_iota(jnp.int32, (B, NB, 1), 1)
        pid = (lax.broadcasted_iota(jnp.int32, (1, 1, PAGES_LOCAL), 2)
               + my_i * PAGES_LOCAL)
        hit = (bt3 == pid) & (kpos < lens3)
        counts = jnp.sum(jnp.where(hit, 1.0, 0.0).astype(jnp.float32),
                         axis=1)
        cnt_keys = jnp.broadcast_to(
            counts[:, :, None], (B, PAGES_LOCAL, BS)
        ).reshape(B, KEYS_LOCAL)

        q = q_ref[:, 0, :, :] * (D ** -0.5)
        k_all = k_ref[:].reshape(KEYS_LOCAL, H, D)
        v_all = v_ref[:].reshape(KEYS_LOCAL, H, D)

        s = lax.dot_general(
            q, k_all, (((2,), (2,)), ((1,), (1,))),
            preferred_element_type=jnp.float32,
        )
        cnt = cnt_keys[None, :, :]
        valid = cnt > 0.0
        m_loc = jnp.max(jnp.where(valid, s, NEG_INF),
                        axis=2, keepdims=True)
        p = jnp.where(valid, cnt * jnp.exp(s - m_loc), 0.0)
        l_loc = jnp.sum(p, axis=2, keepdims=True)
        o_loc = lax.dot_general(
            p, v_all, (((2,), (0,)), ((0,), (1,))),
            preferred_element_type=jnp.float32,
        )

        comm_ref[0, :, :, 0:D] = o_loc
        comm_ref[0, :, :, D:D + 1] = m_loc
        comm_ref[0, :, :, D + 1:D + 2] = l_loc

        barrier_sem = pltpu.get_barrier_semaphore()
        for t in range(1, N_DEV):
            pl.semaphore_signal(
                barrier_sem, inc=1,
                device_id=((my_i + t) % N_DEV,),
                device_id_type=pl.DeviceIdType.MESH,
            )
        pl.semaphore_wait(barrier_sem, N_DEV - 1)

        rdmas = []
        for t in range(1, N_DEV):
            rdma = pltpu.make_async_remote_copy(
                src_ref=comm_ref.at[0],
                dst_ref=comm_ref.at[t],
                send_sem=send_sems.at[t - 1],
                recv_sem=recv_sems.at[t - 1],
                device_id=((my_i + t) % N_DEV,),
                device_id_type=pl.DeviceIdType.MESH,
            )
            rdma.start()
            rdmas.append(rdma)

        acc_o, acc_m, acc_l = o_loc, m_loc, l_loc
        for t in range(1, N_DEV):
            rdmas[t - 1].wait()
            o_in = comm_ref[t, :, :, 0:D]
            m_in = comm_ref[t, :, :, D:D + 1]
            l_in = comm_ref[t, :, :, D + 1:D + 2]
            m_new = jnp.maximum(acc_m, m_in)
            a = jnp.exp(acc_m - m_new)
            bweight = jnp.exp(m_in - m_new)
            acc_o = acc_o * a + o_in * bweight
            acc_l = acc_l * a + l_in * bweight
            acc_m = m_new

        res = acc_o / acc_l
        out_ref[:, 0, :, :] = jnp.transpose(res, (1, 0, 2))

    return pl.pallas_call(
        body,
        out_shape=jax.ShapeDtypeStruct((B, 1, H, D), jnp.float32),
        in_specs=[pl.BlockSpec(memory_space=pltpu.VMEM)] * 5,
        out_specs=pl.BlockSpec(memory_space=pltpu.VMEM),
        scratch_shapes=[
            pltpu.VMEM((N_DEV, H, B, PAGES_LOCAL), jnp.float32),
            pltpu.SemaphoreType.DMA((N_DEV - 1,)),
            pltpu.SemaphoreType.DMA((N_DEV - 1,)),
        ],
        compiler_params=pltpu.CompilerParams(collective_id=0),
    )(Q, K, V, bt, lens2)


# baseline (device time: 65612 ns/iter reference)
import jax
import jax.numpy as jnp
from jax import lax
from jax.experimental import pallas as pl
from jax.experimental.pallas import tpu as pltpu

N_DEV = 4
B, H, D, BS = 16, 16, 64, 16
NB = 128
PAGES_LOCAL = 128
KEYS_LOCAL = PAGES_LOCAL * BS
NEG_INF = -1e30


def kernel(Q, K, V, bt, lens):
    lens2 = lens.reshape(B, 1).astype(jnp.int32)

    def body(q_ref, k_ref, v_ref, bt_ref, lens_ref, out_ref,
             comm_ref, send_sems, recv_sems):
        my_i = lax.axis_index("i")

        bt3 = bt_ref[:, :][:, :, None]
        lens3 = lens_ref[:, :][:, :, None]
        kpos = lax.broadcasted_iota(jnp.int32, (B, NB, 1), 1)
        pid = (lax.broadcasted_iota(jnp.int32, (1, 1, PAGES_LOCAL), 2)
               + my_i * PAGES_LOCAL)
        hit = (bt3 == pid) & (kpos < lens3)
        counts = jnp.sum(jnp.where(hit, 1.0, 0.0).astype(jnp.float32),
                         axis=1)
        cnt_keys = jnp.broadcast_to(
            counts[:, :, None], (B, PAGES_LOCAL, BS)
        ).reshape(B, KEYS_LOCAL)
        valid = cnt_keys > 0.0

        for h in range(H):
            q_h = q_ref[:, 0, h, :] * (D ** -0.5)
            k_h = k_ref[:, :, h, :].reshape(KEYS_LOCAL, D)
            v_h = v_ref[:, :, h, :].reshape(KEYS_LOCAL, D)
            s_h = lax.dot_general(
                q_h, k_h, (((1,), (1,)), ((), ())),
                preferred_element_type=jnp.float32,
            )
            m_h = jnp.max(jnp.where(valid, s_h, NEG_INF),
                          axis=1, keepdims=True)
            p_h = jnp.where(valid, cnt_keys * jnp.exp(s_h - m_h), 0.0)
            l_h = jnp.sum(p_h, axis=1, keepdims=True)
            o_h = lax.dot_general(
                p_h, v_h, (((1,), (0,)), ((), ())),
                preferred_element_type=jnp.float32,
            )
            comm_ref[0, h, :, 0:D] = o_h
            comm_ref[0, h, :, D:D + 1] = m_h
            comm_ref[0, h, :, D + 1:D + 2] = l_h

        barrier_sem = pltpu.get_barrier_semaphore()
        for t in range(1, N_DEV):
            pl.semaphore_signal(
                barrier_sem, inc=1,
                device_id=((my_i + t) % N_DEV,),
                device_id_type=pl.DeviceIdType.MESH,
            )
        pl.semaphore_wait(barrier_sem, N_DEV - 1)

        rdmas = []
        for t in range(1, N_DEV):
            rdma = pltpu.make_async_remote_copy(
                src_ref=comm_ref.at[0],
                dst_ref=comm_ref.at[t],
                send_sem=send_sems.at[t - 1],
                recv_sem=recv_sems.at[t - 1],
                device_id=((my_i + t) % N_DEV,),
                device_id_type=pl.DeviceIdType.MESH,
            )
            rdma.start()
            rdmas.append(rdma)

        acc_o = comm_ref[0, :, :, 0:D]
        acc_m = comm_ref[0, :, :, D:D + 1]
        acc_l = comm_ref[0, :, :, D + 1:D + 2]
        for t in range(1, N_DEV):
            rdmas[t - 1].wait()
            o_in = comm_ref[t, :, :, 0:D]
            m_in = comm_ref[t, :, :, D:D + 1]
            l_in = comm_ref[t, :, :, D + 1:D + 2]
            m_new = jnp.maximum(acc_m, m_in)
            a = jnp.exp(acc_m - m_new)
            bweight = jnp.exp(m_in - m_new)
            acc_o = acc_o * a + o_in * bweight
            acc_l = acc_l * a + l_in * bweight
            acc_m = m_new

        res = acc_o / acc_l
        out_ref[:, 0, :, :] = jnp.transpose(res, (1, 0, 2))

    return pl.pallas_call(
        body,
        out_shape=jax.ShapeDtypeStruct((B, 1, H, D), jnp.float32),
        in_specs=[pl.BlockSpec(memory_space=pltpu.VMEM)] * 5,
        out_specs=pl.BlockSpec(memory_space=pltpu.VMEM),
        scratch_shapes=[
            pltpu.VMEM((N_DEV, H, B, PAGES_LOCAL), jnp.float32),
            pltpu.SemaphoreType.DMA((N_DEV - 1,)),
            pltpu.SemaphoreType.DMA((N_DEV - 1,)),
        ],
        compiler_params=pltpu.CompilerParams(collective_id=0),
    )(Q, K, V, bt, lens2)


# device time: 51461 ns/iter; 1.2750x vs baseline; 1.2750x over previous
import jax
import jax.numpy as jnp
from jax import lax
from jax.experimental import pallas as pl
from jax.experimental.pallas import tpu as pltpu

N_DEV = 4
B, H, D, BS = 16, 16, 64, 16
NB = 128
PAGES_LOCAL = 128
KEYS_LOCAL = PAGES_LOCAL * BS
NEG_INF = -1e30


def kernel(Q, K, V, bt, lens):
    lens2 = lens.reshape(B, 1).astype(jnp.int32)
    qh = Q.reshape(B, H, D).transpose(1, 0, 2)
    kt = K.transpose(2, 3, 0, 1).reshape(H, D, KEYS_LOCAL)
    vt = V.transpose(2, 3, 0, 1).reshape(H, D, KEYS_LOCAL)

    def body(q_ref, k_ref, v_ref, bt_ref, lens_ref, out_ref,
             comm_ref, send_sems, recv_sems):
        my_i = lax.axis_index("i")

        bt3 = bt_ref[:, :][:, :, None]
        lens3 = lens_ref[:, :][:, :, None]
        kpos = lax.broadcasted_iota(jnp.int32, (B, NB, 1), 1)
        pid = (lax.broadcasted_iota(jnp.int32, (1, 1, PAGES_LOCAL), 2)
               + my_i * PAGES_LOCAL)
        hit = (bt3 == pid) & (kpos < lens3)
        counts = jnp.sum(jnp.where(hit, 1.0, 0.0).astype(jnp.float32),
                         axis=1)
        cnt_keys = jnp.broadcast_to(
            counts[:, :, None], (B, PAGES_LOCAL, BS)
        ).reshape(B, KEYS_LOCAL)
        valid = cnt_keys > 0.0

        for h in range(H):
            q_h = q_ref[h] * (D ** -0.5)
            k_h = k_ref[h]
            v_h = v_ref[h]
            s_h = lax.dot_general(
                q_h, k_h, (((1,), (0,)), ((), ())),
                preferred_element_type=jnp.float32,
            )
            m_h = jnp.max(jnp.where(valid, s_h, NEG_INF),
                          axis=1, keepdims=True)
            p_h = jnp.where(valid, cnt_keys * jnp.exp(s_h - m_h), 0.0)
            l_h = jnp.sum(p_h, axis=1, keepdims=True)
            o_h = lax.dot_general(
                p_h, v_h, (((1,), (1,)), ((), ())),
                preferred_element_type=jnp.float32,
            )
            comm_ref[0, h, :, 0:D] = o_h
            comm_ref[0, h, :, D:D + 1] = m_h
            comm_ref[0, h, :, D + 1:D + 2] = l_h

        barrier_sem = pltpu.get_barrier_semaphore()
        for t in range(1, N_DEV):
            pl.semaphore_signal(
                barrier_sem, inc=1,
                device_id=((my_i + t) % N_DEV,),
                device_id_type=pl.DeviceIdType.MESH,
            )
        pl.semaphore_wait(barrier_sem, N_DEV - 1)

        rdmas = []
        for t in range(1, N_DEV):
            rdma = pltpu.make_async_remote_copy(
                src_ref=comm_ref.at[0],
                dst_ref=comm_ref.at[t],
                send_sem=send_sems.at[t - 1],
                recv_sem=recv_sems.at[t - 1],
                device_id=((my_i + t) % N_DEV,),
                device_id_type=pl.DeviceIdType.MESH,
            )
            rdma.start()
            rdmas.append(rdma)

        acc_o = comm_ref[0, :, :, 0:D]
        acc_m = comm_ref[0, :, :, D:D + 1]
        acc_l = comm_ref[0, :, :, D + 1:D + 2]
        for t in range(1, N_DEV):
            rdmas[t - 1].wait()
            o_in = comm_ref[t, :, :, 0:D]
            m_in = comm_ref[t, :, :, D:D + 1]
            l_in = comm_ref[t, :, :, D + 1:D + 2]
            m_new = jnp.maximum(acc_m, m_in)
            a = jnp.exp(acc_m - m_new)
            bweight = jnp.exp(m_in - m_new)
            acc_o = acc_o * a + o_in * bweight
            acc_l = acc_l * a + l_in * bweight
            acc_m = m_new

        res = acc_o / acc_l
        out_ref[:, 0, :, :] = jnp.transpose(res, (1, 0, 2))

    return pl.pallas_call(
        body,
        out_shape=jax.ShapeDtypeStruct((B, 1, H, D), jnp.float32),
        in_specs=[pl.BlockSpec(memory_space=pltpu.VMEM)] * 5,
        out_specs=pl.BlockSpec(memory_space=pltpu.VMEM),
        scratch_shapes=[
            pltpu.VMEM((N_DEV, H, B, PAGES_LOCAL), jnp.float32),
            pltpu.SemaphoreType.DMA((N_DEV - 1,)),
            pltpu.SemaphoreType.DMA((N_DEV - 1,)),
        ],
        compiler_params=pltpu.CompilerParams(collective_id=0),
    )(qh, kt, vt, bt, lens2)
